# baseline (device time: 35352 ns/iter reference)
import jax
import jax.numpy as jnp
from jax import lax
from jax.experimental import pallas as pl
from jax.experimental.pallas import tpu as pltpu


def kernel(Q, K, V):
    b, sq, h, d = Q.shape
    skv = K.shape[1]

    def body(q_ref, k_ref, v_ref, out_ref, kbuf, vbuf, sems):
        copies = []
        for bb in range(b):
            ck = pltpu.make_async_copy(k_ref.at[bb], kbuf.at[bb], sems.at[bb])
            cv = pltpu.make_async_copy(v_ref.at[bb], vbuf.at[bb], sems.at[b + bb])
            ck.start()
            cv.start()
            copies.append((ck, cv))
        for ck, cv in copies:
            ck.wait()
            cv.wait()
        out_ref[...] = (
            q_ref[...]
            + kbuf[0:b, 0:sq]
            + vbuf[0:b, 0:sq]
        )

    return pl.pallas_call(
        body,
        out_shape=jax.ShapeDtypeStruct((b, sq, h, d), jnp.float32),
        in_specs=[
            pl.BlockSpec(memory_space=pltpu.VMEM),
            pl.BlockSpec(memory_space=pltpu.MemorySpace.HBM),
            pl.BlockSpec(memory_space=pltpu.MemorySpace.HBM),
        ],
        out_specs=pl.BlockSpec(memory_space=pltpu.VMEM),
        scratch_shapes=[
            pltpu.VMEM((8, skv, h, d), jnp.float32),
            pltpu.VMEM((8, skv, h, d), jnp.float32),
            pltpu.SemaphoreType.DMA((16,)),
        ],
        compiler_params=pltpu.CompilerParams(
            vmem_limit_bytes=100 * 1024 * 1024,
        ),
    )(Q, K, V)


# device time: 17878 ns/iter; 1.9774x vs baseline; 1.9774x over previous
import jax
import jax.numpy as jnp
from jax import lax
from jax.experimental import pallas as pl
from jax.experimental.pallas import tpu as pltpu

N_DEV = 4


def kernel(Q, K, V):
    b, sq, h, d = Q.shape
    skv = K.shape[1]
    scale = d ** -0.5

    q2 = Q.reshape(b, h, d)
    kt = jnp.transpose(K, (0, 2, 3, 1))
    vt = jnp.transpose(V, (0, 2, 3, 1))

    def body(q_ref, k_ref, v_ref, out_ref,
             commU, commML, send_u, recv_u, send_ml, recv_ml):
        my = lax.axis_index("i")

        barrier_sem = pltpu.get_barrier_semaphore()
        for off in (1, 2, 3):
            nbr = lax.rem(my + off, N_DEV)
            pl.semaphore_signal(
                barrier_sem, inc=1,
                device_id=(nbr,), device_id_type=pl.DeviceIdType.MESH,
            )
        pl.semaphore_wait(barrier_sem, N_DEV - 1)

        for bb in range(b):
            qb = q_ref[bb]
            kb = k_ref[bb]
            vb = v_ref[bb]
            s = jnp.sum(kb * qb[:, :, None], axis=1) * scale
            m = jnp.max(s, axis=-1, keepdims=True)
            p = jnp.exp(s - m)
            l = jnp.sum(p, axis=-1, keepdims=True)
            u = jnp.sum(p[:, None, :] * vb, axis=-1)
            commU[0, bb] = u
            commML[0, 0, bb] = m
            commML[0, 1, bb] = l

        rdmas = []
        for off in (1, 2, 3):
            dst = lax.rem(my + off, N_DEV)
            slot = N_DEV - off
            ru = pltpu.make_async_remote_copy(
                src_ref=commU.at[0],
                dst_ref=commU.at[slot],
                send_sem=send_u.at[off - 1],
                recv_sem=recv_u.at[slot - 1],
                device_id=(dst,),
                device_id_type=pl.DeviceIdType.MESH,
            )
            rml = pltpu.make_async_remote_copy(
                src_ref=commML.at[0],
                dst_ref=commML.at[slot],
                send_sem=send_ml.at[off - 1],
                recv_sem=recv_ml.at[slot - 1],
                device_id=(dst,),
                device_id_type=pl.DeviceIdType.MESH,
            )
            ru.start()
            rml.start()
            rdmas.append((ru, rml))

        for ru, rml in rdmas:
            ru.wait_send()
            rml.wait_send()
            ru.wait_recv()
            rml.wait_recv()

        u_run = commU[0]
        m_run = commML[0, 0]
        l_run = commML[0, 1]
        for s in (1, 2, 3):
            u_s = commU[s]
            m_s = commML[s, 0]
            l_s = commML[s, 1]
            m_new = jnp.maximum(m_run, m_s)
            a = jnp.exp(m_run - m_new)
            c = jnp.exp(m_s - m_new)
            l_run = l_run * a + l_s * c
            u_run = u_run * a + u_s * c
            m_run = m_new

        out_ref[...] = (u_run / l_run).reshape(b, sq, h, d)

    return pl.pallas_call(
        body,
        out_shape=jax.ShapeDtypeStruct((b, sq, h, d), jnp.float32),
        in_specs=[pl.BlockSpec(memory_space=pltpu.VMEM)] * 3,
        out_specs=pl.BlockSpec(memory_space=pltpu.VMEM),
        scratch_shapes=[
            pltpu.VMEM((N_DEV, b, h, d), jnp.float32),
            pltpu.VMEM((N_DEV, 2, b, h, 1), jnp.float32),
            pltpu.SemaphoreType.DMA((3,)),
            pltpu.SemaphoreType.DMA((3,)),
            pltpu.SemaphoreType.DMA((3,)),
            pltpu.SemaphoreType.DMA((3,)),
        ],
        compiler_params=pltpu.CompilerParams(
            collective_id=0,
            vmem_limit_bytes=100 * 1024 * 1024,
        ),
    )(q2, kt, vt)


# device time: 15343 ns/iter; 2.3041x vs baseline; 1.1652x over previous
import jax
import jax.numpy as jnp
from jax import lax
from jax.experimental import pallas as pl
from jax.experimental.pallas import tpu as pltpu

N_DEV = 4


def kernel(Q, K, V):
    b, sq, h, d = Q.shape
    skv = K.shape[1]
    scale = d ** -0.5

    q2 = Q.reshape(b, h, d)
    kt = jnp.transpose(K, (0, 2, 3, 1))
    vt = jnp.transpose(V, (0, 2, 3, 1))

    def body(q_ref, k_ref, v_ref, out_ref,
             commU, commML, kbuf, vbuf, load_sems,
             send_u, recv_u, send_ml, recv_ml):
        my = lax.axis_index("i")

        loads = []
        for bb in range(b):
            ck = pltpu.make_async_copy(
                k_ref.at[bb], kbuf.at[bb], load_sems.at[bb])
            cv = pltpu.make_async_copy(
                v_ref.at[bb], vbuf.at[bb], load_sems.at[b + bb])
            ck.start()
            cv.start()
            loads.append((ck, cv))

        barrier_sem = pltpu.get_barrier_semaphore()
        for off in (1, 2, 3):
            nbr = lax.rem(my + off, N_DEV)
            pl.semaphore_signal(
                barrier_sem, inc=1,
                device_id=(nbr,), device_id_type=pl.DeviceIdType.MESH,
            )
        pl.semaphore_wait(barrier_sem, N_DEV - 1)

        for bb in range(b):
            ck, cv = loads[bb]
            ck.wait()
            cv.wait()
            qb = q_ref[bb]
            kb = kbuf[bb]
            vb = vbuf[bb]
            s = jnp.sum(kb * qb[:, :, None], axis=1) * scale
            m = jnp.max(s, axis=-1, keepdims=True)
            p = jnp.exp(s - m)
            l = jnp.sum(p, axis=-1, keepdims=True)
            u = jnp.sum(p[:, None, :] * vb, axis=-1)
            commU[0, bb] = u
            commML[0, 0, bb] = m
            commML[0, 1, bb] = l

        rdmas = []
        for off in (1, 2, 3):
            dst = lax.rem(my + off, N_DEV)
            slot = N_DEV - off
            ru = pltpu.make_async_remote_copy(
                src_ref=commU.at[0],
                dst_ref=commU.at[slot],
                send_sem=send_u.at[off - 1],
                recv_sem=recv_u.at[slot - 1],
                device_id=(dst,),
                device_id_type=pl.DeviceIdType.MESH,
            )
            rml = pltpu.make_async_remote_copy(
                src_ref=commML.at[0],
                dst_ref=commML.at[slot],
                send_sem=send_ml.at[off - 1],
                recv_sem=recv_ml.at[slot - 1],
                device_id=(dst,),
                device_id_type=pl.DeviceIdType.MESH,
            )
            ru.start()
            rml.start()
            rdmas.append((ru, rml))

        for ru, rml in rdmas:
            ru.wait_send()
            rml.wait_send()
            ru.wait_recv()
            rml.wait_recv()

        u_run = commU[0]
        m_run = commML[0, 0]
        l_run = commML[0, 1]
        for s in (1, 2, 3):
            u_s = commU[s]
            m_s = commML[s, 0]
            l_s = commML[s, 1]
            m_new = jnp.maximum(m_run, m_s)
            a = jnp.exp(m_run - m_new)
            c = jnp.exp(m_s - m_new)
            l_run = l_run * a + l_s * c
            u_run = u_run * a + u_s * c
            m_run = m_new

        out_ref[...] = (u_run / l_run).reshape(b, sq, h, d)

    return pl.pallas_call(
        body,
        out_shape=jax.ShapeDtypeStruct((b, sq, h, d), jnp.float32),
        in_specs=[
            pl.BlockSpec(memory_space=pltpu.VMEM),
            pl.BlockSpec(memory_space=pltpu.MemorySpace.HBM),
            pl.BlockSpec(memory_space=pltpu.MemorySpace.HBM),
        ],
        out_specs=pl.BlockSpec(memory_space=pltpu.VMEM),
        scratch_shapes=[
            pltpu.VMEM((N_DEV, b, h, d), jnp.float32),
            pltpu.VMEM((N_DEV, 2, b, h, 1), jnp.float32),
            pltpu.VMEM((b, h, d, skv), jnp.float32),
            pltpu.VMEM((b, h, d, skv), jnp.float32),
            pltpu.SemaphoreType.DMA((2 * b,)),
            pltpu.SemaphoreType.DMA((3,)),
            pltpu.SemaphoreType.DMA((3,)),
            pltpu.SemaphoreType.DMA((3,)),
            pltpu.SemaphoreType.DMA((3,)),
        ],
        compiler_params=pltpu.CompilerParams(
            collective_id=0,
            vmem_limit_bytes=100 * 1024 * 1024,
        ),
    )(q2, kt, vt)


# device time: 8530 ns/iter; 4.1444x vs baseline; 1.7987x over previous
import jax
import jax.numpy as jnp
from jax import lax
from jax.experimental import pallas as pl
from jax.experimental.pallas import tpu as pltpu

N_DEV = 4


def kernel(Q, K, V):
    b, sq, h, d = Q.shape
    skv = K.shape[1]
    scale = d ** -0.5

    q2 = Q.reshape(b, h, d)
    kt = jnp.transpose(K, (0, 2, 3, 1))
    vt = jnp.transpose(V, (0, 2, 3, 1))

    def body(q_ref, k_ref, v_ref, out_ref,
             commU, commML, kbuf, vbuf, load_sems,
             send_u, recv_u, send_ml, recv_ml):
        my = lax.axis_index("i")

        loads = []
        for bb in range(b):
            ck = pltpu.make_async_copy(
                k_ref.at[bb], kbuf.at[bb], load_sems.at[bb])
            cv = pltpu.make_async_copy(
                v_ref.at[bb], vbuf.at[bb], load_sems.at[b + bb])
            ck.start()
            cv.start()
            loads.append((ck, cv))

        barrier_sem = pltpu.get_barrier_semaphore()
        for off in (1, 2, 3):
            nbr = lax.rem(my + off, N_DEV)
            pl.semaphore_signal(
                barrier_sem, inc=1,
                device_id=(nbr,), device_id_type=pl.DeviceIdType.MESH,
            )
        pl.semaphore_wait(barrier_sem, N_DEV - 1)

        for bb in range(b):
            ck, cv = loads[bb]
            ck.wait()
            cv.wait()
            qb = q_ref[bb].astype(jnp.bfloat16)
            kb = kbuf[bb].astype(jnp.bfloat16)
            vb = vbuf[bb].astype(jnp.bfloat16)
            s = lax.dot_general(
                qb, kb, (((1,), (1,)), ((0,), (0,))),
                preferred_element_type=jnp.float32) * scale
            m = jnp.max(s, axis=-1, keepdims=True)
            p = jnp.exp(s - m)
            l = jnp.sum(p, axis=-1, keepdims=True)
            u = lax.dot_general(
                p.astype(jnp.bfloat16), vb, (((1,), (2,)), ((0,), (0,))),
                preferred_element_type=jnp.float32)
            commU[0, bb] = u
            commML[0, 0, bb] = m
            commML[0, 1, bb] = l

        rdmas = []
        for off in (1, 2, 3):
            dst = lax.rem(my + off, N_DEV)
            slot = N_DEV - off
            ru = pltpu.make_async_remote_copy(
                src_ref=commU.at[0],
                dst_ref=commU.at[slot],
                send_sem=send_u.at[off - 1],
                recv_sem=recv_u.at[slot - 1],
                device_id=(dst,),
                device_id_type=pl.DeviceIdType.MESH,
            )
            rml = pltpu.make_async_remote_copy(
                src_ref=commML.at[0],
                dst_ref=commML.at[slot],
                send_sem=send_ml.at[off - 1],
                recv_sem=recv_ml.at[slot - 1],
                device_id=(dst,),
                device_id_type=pl.DeviceIdType.MESH,
            )
            ru.start()
            rml.start()
            rdmas.append((ru, rml))

        for ru, rml in rdmas:
            ru.wait_send()
            rml.wait_send()
            ru.wait_recv()
            rml.wait_recv()

        u_run = commU[0]
        m_run = commML[0, 0]
        l_run = commML[0, 1]
        for s in (1, 2, 3):
            u_s = commU[s]
            m_s = commML[s, 0]
            l_s = commML[s, 1]
            m_new = jnp.maximum(m_run, m_s)
            a = jnp.exp(m_run - m_new)
            c = jnp.exp(m_s - m_new)
            l_run = l_run * a + l_s * c
            u_run = u_run * a + u_s * c
            m_run = m_new

        out_ref[...] = (u_run / l_run).reshape(b, sq, h, d)

    return pl.pallas_call(
        body,
        out_shape=jax.ShapeDtypeStruct((b, sq, h, d), jnp.float32),
        in_specs=[
            pl.BlockSpec(memory_space=pltpu.VMEM),
            pl.BlockSpec(memory_space=pltpu.MemorySpace.HBM),
            pl.BlockSpec(memory_space=pltpu.MemorySpace.HBM),
        ],
        out_specs=pl.BlockSpec(memory_space=pltpu.VMEM),
        scratch_shapes=[
            pltpu.VMEM((N_DEV, b, h, d), jnp.float32),
            pltpu.VMEM((N_DEV, 2, b, h, 1), jnp.float32),
            pltpu.VMEM((b, h, d, skv), jnp.float32),
            pltpu.VMEM((b, h, d, skv), jnp.float32),
            pltpu.SemaphoreType.DMA((2 * b,)),
            pltpu.SemaphoreType.DMA((3,)),
            pltpu.SemaphoreType.DMA((3,)),
            pltpu.SemaphoreType.DMA((3,)),
            pltpu.SemaphoreType.DMA((3,)),
        ],
        compiler_params=pltpu.CompilerParams(
            collective_id=0,
            vmem_limit_bytes=100 * 1024 * 1024,
        ),
    )(q2, kt, vt)
